# baseline (speedup 1.0000x reference)
import functools

import jax
import jax.numpy as jnp
import numpy as np
from jax.experimental import pallas as pl


def _prep_kernel(s_ref, n_ref, minv_ref):
    S = s_ref[...]
    n = n_ref[...]
    d = S.shape[0]
    ii = jax.lax.broadcasted_iota(jnp.int32, S.shape, 0)
    kk = jax.lax.broadcasted_iota(jnp.int32, S.shape, 1)
    eye3 = (ii == kk).astype(S.dtype)
    A = S / n[None, :, :] + 1e-6 * eye3
    Inv = eye3
    for j in range(d):
        rowA = A[j]
        inv_piv = 1.0 / rowA[j]
        rowA = rowA * inv_piv[None, :]
        rowI = Inv[j] * inv_piv[None, :]
        colA = A[:, j, :]
        is_row_j = ii == j
        A = jnp.where(is_row_j, rowA[None, :, :],
                      A - colA[:, None, :] * rowA[None, :, :])
        Inv = jnp.where(is_row_j, rowI[None, :, :],
                        Inv - colA[:, None, :] * rowI[None, :, :])
    minv_ref[...] = Inv


def _d2_kernel(xt_ref, mut_ref, bd_ref, d2_ref):
    g = 4
    xT = xt_ref[...]
    mu_blk = mut_ref[...][0]
    bd = bd_ref[...]
    n_groups = bd.shape[0]
    gd = bd.shape[1]
    bb = xT.shape[1]
    xt4 = jnp.concatenate([xT] * g, axis=0)
    cols = []
    for q in range(n_groups):
        diffT = xt4 - mu_blk[:, q][:, None]
        diffb = diffT.astype(jnp.bfloat16)
        tT = jnp.dot(bd[q], diffb, preferred_element_type=jnp.float32)
        prodT = tT * diffT
        d2q = jnp.sum(prodT.reshape(g, gd // g, bb), axis=1)
        cols.append(d2q)
    out = jnp.concatenate(cols, axis=0)
    d2_ref[...] = out.T


def _defuzz_kernel(d2_ref, lab_ref, ls_ref, preds_ref, clus_ref):
    d2 = d2_ref[...]
    Gamma = jnp.exp(-0.5 * d2)
    Gn = Gamma / (jnp.sum(Gamma, axis=1, keepdims=True) + 1e-12)
    ls = jnp.dot(Gn.astype(jnp.bfloat16), lab_ref[...],
                 preferred_element_type=jnp.float32)
    ls = ls / (jnp.sum(ls, axis=1, keepdims=True) + 1e-12)
    ls_ref[...] = ls
    preds_ref[...] = jnp.argmax(ls, axis=1).astype(jnp.int32)[:, None]
    clus_ref[...] = jnp.argmax(Gamma, axis=1).astype(jnp.int32)[:, None]


@functools.partial(jax.jit, static_argnames=())
def kernel(data, mu, S, n, cluster_labels):
    B, D = data.shape
    C = mu.shape[0]
    NC = cluster_labels.shape[1]
    g = 4
    gd = g * D

    cb = 128
    minv_t = pl.pallas_call(
        _prep_kernel,
        grid=(C // cb,),
        in_specs=[
            pl.BlockSpec((D, D, cb), lambda i: (0, 0, i)),
            pl.BlockSpec((1, cb), lambda i: (0, i)),
        ],
        out_specs=pl.BlockSpec((D, D, cb), lambda i: (0, 0, i)),
        out_shape=jax.ShapeDtypeStruct((D, D, C), jnp.float32),
    )(S.transpose(1, 2, 0), n.reshape(1, C))

    minv_bf = minv_t.transpose(2, 0, 1).astype(jnp.bfloat16)
    tmp = minv_bf.reshape(C // g, g, D, D)
    eye_g = jnp.eye(g, dtype=jnp.bfloat16)
    bd = (tmp[:, :, :, None, :] * eye_g[None, :, None, :, None]
          ).reshape(C // g, gd, gd)
    CB = 128
    mut = mu.reshape(C // g, gd).T.reshape(gd, C // CB, CB // g).transpose(1, 0, 2)
    dataT = data.T
    lab_bf = cluster_labels.astype(jnp.bfloat16)

    bb = 512
    d2 = pl.pallas_call(
        _d2_kernel,
        grid=(B // bb, C // CB),
        in_specs=[
            pl.BlockSpec((D, bb), lambda i, j: (0, i)),
            pl.BlockSpec((1, gd, CB // g), lambda i, j: (j, 0, 0)),
            pl.BlockSpec((CB // g, gd, gd), lambda i, j: (j, 0, 0)),
        ],
        out_specs=pl.BlockSpec((bb, CB), lambda i, j: (i, j)),
        out_shape=jax.ShapeDtypeStruct((B, C), jnp.float32),
    )(dataT, mut, bd)

    ls, preds, clus = pl.pallas_call(
        _defuzz_kernel,
        grid=(B // bb,),
        in_specs=[
            pl.BlockSpec((bb, C), lambda i: (i, 0)),
            pl.BlockSpec((C, NC), lambda i: (0, 0)),
        ],
        out_specs=[
            pl.BlockSpec((bb, NC), lambda i: (i, 0)),
            pl.BlockSpec((bb, 1), lambda i: (i, 0)),
            pl.BlockSpec((bb, 1), lambda i: (i, 0)),
        ],
        out_shape=[
            jax.ShapeDtypeStruct((B, NC), jnp.float32),
            jax.ShapeDtypeStruct((B, 1), jnp.int32),
            jax.ShapeDtypeStruct((B, 1), jnp.int32),
        ],
    )(d2, lab_bf)

    return ls, preds.reshape(B), clus.reshape(B)


# speedup vs baseline: 1.0623x; 1.0623x over previous
import functools

import jax
import jax.numpy as jnp
from jax.experimental import pallas as pl
from jax.experimental.pallas import tpu as pltpu


def _prep_kernel(s_ref, n_ref, minv_ref):
    S = s_ref[...]
    n = n_ref[...]
    d = S.shape[0]
    ii = jax.lax.broadcasted_iota(jnp.int32, S.shape, 0)
    kk = jax.lax.broadcasted_iota(jnp.int32, S.shape, 1)
    eye3 = (ii == kk).astype(S.dtype)
    A = S / n[None, :, :] + 1e-6 * eye3
    Inv = eye3
    for j in range(d):
        rowA = A[j]
        inv_piv = 1.0 / rowA[j]
        rowA = rowA * inv_piv[None, :]
        rowI = Inv[j] * inv_piv[None, :]
        colA = A[:, j, :]
        is_row_j = ii == j
        A = jnp.where(is_row_j, rowA[None, :, :],
                      A - colA[:, None, :] * rowA[None, :, :])
        Inv = jnp.where(is_row_j, rowI[None, :, :],
                        Inv - colA[:, None, :] * rowI[None, :, :])
    minv_ref[...] = Inv


def _fused_kernel(xt_ref, mut_ref, bd_ref, labt_ref, lst_ref, preds_ref,
                  clus_ref, g_scr):
    g = 4
    xT = xt_ref[...]
    bb = xT.shape[1]
    n_chunks = mut_ref.shape[0]
    n_groups = mut_ref.shape[2]
    gd = mut_ref.shape[1]
    CB = n_groups * g
    xt4 = jnp.concatenate([xT] * g, axis=0)
    big = jnp.int32(2 ** 30)
    run_max = jnp.full((1, bb), -jnp.inf, dtype=jnp.float32)
    run_arg = jnp.zeros((1, bb), dtype=jnp.int32)
    ssum = jnp.zeros((1, bb), dtype=jnp.float32)
    idx_c = jax.lax.broadcasted_iota(jnp.int32, (CB, bb), 0)
    for jc in range(n_chunks):
        mu_blk = mut_ref[jc][...]
        cols = []
        for q in range(n_groups):
            diffT = xt4 - mu_blk[:, q][:, None]
            diffb = diffT.astype(jnp.bfloat16)
            tT = jnp.dot(bd_ref[jc * n_groups + q][...], diffb,
                         preferred_element_type=jnp.float32)
            prodT = tT * diffT
            cols.append(jnp.sum(prodT.reshape(g, gd // g, bb), axis=1))
        d2c = jnp.concatenate(cols, axis=0)
        Gc = jnp.exp(-0.5 * d2c)
        g_scr[jc * CB:(jc + 1) * CB, :] = Gc
        ssum = ssum + jnp.sum(Gc, axis=0, keepdims=True)
        cmax = jnp.max(Gc, axis=0, keepdims=True)
        carg = jnp.min(jnp.where(Gc == cmax, idx_c + jc * CB, big),
                       axis=0, keepdims=True)
        upd = cmax > run_max
        run_arg = jnp.where(upd, carg, run_arg)
        run_max = jnp.maximum(run_max, cmax)
    s = ssum + 1e-12
    labt = labt_ref[...]
    nc = labt.shape[0]
    gnb = (g_scr[...] / s).astype(jnp.bfloat16)
    lsT = jnp.dot(labt, gnb, preferred_element_type=jnp.float32)
    lsT = lsT / (jnp.sum(lsT, axis=0, keepdims=True) + 1e-12)
    lst_ref[...] = lsT
    pmax = jnp.max(lsT, axis=0, keepdims=True)
    idx_p = jax.lax.broadcasted_iota(jnp.int32, (nc, bb), 0)
    preds_ref[...] = jnp.min(jnp.where(lsT == pmax, idx_p, big),
                             axis=0, keepdims=True)
    clus_ref[...] = run_arg


@functools.partial(jax.jit, static_argnames=())
def kernel(data, mu, S, n, cluster_labels):
    B, D = data.shape
    C = mu.shape[0]
    NC = cluster_labels.shape[1]
    g = 4
    gd = g * D

    cb = 128
    minv_t = pl.pallas_call(
        _prep_kernel,
        grid=(C // cb,),
        in_specs=[
            pl.BlockSpec((D, D, cb), lambda i: (0, 0, i)),
            pl.BlockSpec((1, cb), lambda i: (0, i)),
        ],
        out_specs=pl.BlockSpec((D, D, cb), lambda i: (0, 0, i)),
        out_shape=jax.ShapeDtypeStruct((D, D, C), jnp.float32),
    )(S.transpose(1, 2, 0), n.reshape(1, C))

    minv_bf = minv_t.transpose(2, 0, 1).astype(jnp.bfloat16)
    tmp = minv_bf.reshape(C // g, g, D, D)
    eye_g = jnp.eye(g, dtype=jnp.bfloat16)
    bd = (tmp[:, :, :, None, :] * eye_g[None, :, None, :, None]
          ).reshape(C // g, gd, gd)
    CB = 128
    mut = mu.reshape(C // g, gd).T.reshape(gd, C // CB, CB // g).transpose(1, 0, 2)
    dataT = data.T
    labt_bf = cluster_labels.T.astype(jnp.bfloat16)

    bb = 512
    lsT, preds, clus = pl.pallas_call(
        _fused_kernel,
        grid=(B // bb,),
        in_specs=[
            pl.BlockSpec((D, bb), lambda i: (0, i)),
            pl.BlockSpec((C // CB, gd, CB // g), lambda i: (0, 0, 0)),
            pl.BlockSpec((C // g, gd, gd), lambda i: (0, 0, 0)),
            pl.BlockSpec((NC, C), lambda i: (0, 0)),
        ],
        out_specs=[
            pl.BlockSpec((NC, bb), lambda i: (0, i)),
            pl.BlockSpec((1, bb), lambda i: (0, i)),
            pl.BlockSpec((1, bb), lambda i: (0, i)),
        ],
        out_shape=[
            jax.ShapeDtypeStruct((NC, B), jnp.float32),
            jax.ShapeDtypeStruct((1, B), jnp.int32),
            jax.ShapeDtypeStruct((1, B), jnp.int32),
        ],
        scratch_shapes=[pltpu.VMEM((C, bb), jnp.float32)],
    )(dataT, mut, bd, labt_bf)

    return lsT.T, preds.reshape(B), clus.reshape(B)


# speedup vs baseline: 1.3783x; 1.2975x over previous
import functools

import jax
import jax.numpy as jnp
from jax.experimental import pallas as pl
from jax.experimental.pallas import tpu as pltpu


def _prep_kernel(s_ref, n_ref, minv_ref):
    S = s_ref[...]
    n = n_ref[...]
    d = S.shape[0]
    ii = jax.lax.broadcasted_iota(jnp.int32, S.shape, 0)
    kk = jax.lax.broadcasted_iota(jnp.int32, S.shape, 1)
    eye3 = (ii == kk).astype(S.dtype)
    A = S / n[None, :, :] + 1e-6 * eye3
    Inv = eye3
    for j in range(d):
        rowA = A[j]
        inv_piv = 1.0 / rowA[j]
        rowA = rowA * inv_piv[None, :]
        rowI = Inv[j] * inv_piv[None, :]
        colA = A[:, j, :]
        is_row_j = ii == j
        A = jnp.where(is_row_j, rowA[None, :, :],
                      A - colA[:, None, :] * rowA[None, :, :])
        Inv = jnp.where(is_row_j, rowI[None, :, :],
                        Inv - colA[:, None, :] * rowI[None, :, :])
    minv_ref[...] = Inv.astype(jnp.bfloat16)


def _fused_kernel(xt_ref, mut_ref, cmp_ref, labt_ref, lst_ref, preds_ref,
                  clus_ref, g_scr):
    g = 4
    xT = xt_ref[...]
    bb = xT.shape[1]
    n_chunks = mut_ref.shape[0]
    n_groups = mut_ref.shape[2]
    gd = mut_ref.shape[1]
    CB = n_groups * g
    xt4 = jnp.concatenate([xT] * g, axis=0)
    rr = jax.lax.broadcasted_iota(jnp.int32, (gd, gd), 0) // (gd // g)
    cc = jax.lax.broadcasted_iota(jnp.int32, (gd, gd), 1) // (gd // g)
    bd_mask = rr == cc
    big = jnp.int32(2 ** 30)
    run_max = jnp.full((1, bb), -jnp.inf, dtype=jnp.float32)
    run_arg = jnp.zeros((1, bb), dtype=jnp.int32)
    ssum = jnp.zeros((1, bb), dtype=jnp.float32)
    idx_c = jax.lax.broadcasted_iota(jnp.int32, (CB, bb), 0)
    for jc in range(n_chunks):
        mu_blk = mut_ref[jc][...]
        cols = []
        for q in range(n_groups):
            diffT = xt4 - mu_blk[:, q][:, None]
            diffb = diffT.astype(jnp.bfloat16)
            cq = cmp_ref[jc * n_groups + q][...]
            bdq = jnp.where(bd_mask, jnp.concatenate([cq] * g, axis=1),
                            jnp.bfloat16(0))
            tT = jnp.dot(bdq, diffb, preferred_element_type=jnp.float32)
            prodT = tT * diffT
            cols.append(jnp.sum(prodT.reshape(g, gd // g, bb), axis=1))
        d2c = jnp.concatenate(cols, axis=0)
        Gc = jnp.exp(-0.5 * d2c)
        g_scr[jc * CB:(jc + 1) * CB, :] = Gc
        ssum = ssum + jnp.sum(Gc, axis=0, keepdims=True)
        cmax = jnp.max(Gc, axis=0, keepdims=True)
        carg = jnp.min(jnp.where(Gc == cmax, idx_c + jc * CB, big),
                       axis=0, keepdims=True)
        upd = cmax > run_max
        run_arg = jnp.where(upd, carg, run_arg)
        run_max = jnp.maximum(run_max, cmax)
    s = ssum + 1e-12
    labt = labt_ref[...]
    nc = labt.shape[0]
    gnb = (g_scr[...] / s).astype(jnp.bfloat16)
    lsT = jnp.dot(labt, gnb, preferred_element_type=jnp.float32)
    lsT = lsT / (jnp.sum(lsT, axis=0, keepdims=True) + 1e-12)
    lst_ref[...] = lsT
    pmax = jnp.max(lsT, axis=0, keepdims=True)
    idx_p = jax.lax.broadcasted_iota(jnp.int32, (nc, bb), 0)
    preds_ref[...] = jnp.min(jnp.where(lsT == pmax, idx_p, big),
                             axis=0, keepdims=True)
    clus_ref[...] = run_arg


@functools.partial(jax.jit, static_argnames=())
def kernel(data, mu, S, n, cluster_labels):
    B, D = data.shape
    C = mu.shape[0]
    NC = cluster_labels.shape[1]
    g = 4
    gd = g * D

    cb = 128
    minv_t = pl.pallas_call(
        _prep_kernel,
        grid=(C // cb,),
        in_specs=[
            pl.BlockSpec((D, D, cb), lambda i: (0, 0, i)),
            pl.BlockSpec((1, cb), lambda i: (0, i)),
        ],
        out_specs=pl.BlockSpec((D, D, cb), lambda i: (0, 0, i)),
        out_shape=jax.ShapeDtypeStruct((D, D, C), jnp.bfloat16),
    )(S.transpose(1, 2, 0), n.reshape(1, C))

    cmp = minv_t.transpose(2, 0, 1).reshape(C // g, gd, D)
    CB = 128
    mut = mu.reshape(C // g, gd).T.reshape(gd, C // CB, CB // g).transpose(1, 0, 2)
    dataT = data.T
    labt_bf = cluster_labels.T.astype(jnp.bfloat16)

    bb = 512
    lsT, preds, clus = pl.pallas_call(
        _fused_kernel,
        grid=(B // bb,),
        in_specs=[
            pl.BlockSpec((D, bb), lambda i: (0, i)),
            pl.BlockSpec((C // CB, gd, CB // g), lambda i: (0, 0, 0)),
            pl.BlockSpec((C // g, gd, D), lambda i: (0, 0, 0)),
            pl.BlockSpec((NC, C), lambda i: (0, 0)),
        ],
        out_specs=[
            pl.BlockSpec((NC, bb), lambda i: (0, i)),
            pl.BlockSpec((1, bb), lambda i: (0, i)),
            pl.BlockSpec((1, bb), lambda i: (0, i)),
        ],
        out_shape=[
            jax.ShapeDtypeStruct((NC, B), jnp.float32),
            jax.ShapeDtypeStruct((1, B), jnp.int32),
            jax.ShapeDtypeStruct((1, B), jnp.int32),
        ],
        scratch_shapes=[pltpu.VMEM((C, bb), jnp.float32)],
    )(dataT, mut, cmp, labt_bf)

    return lsT.T, preds.reshape(B), clus.reshape(B)


# speedup vs baseline: 1.6307x; 1.1831x over previous
import functools

import jax
import jax.numpy as jnp
from jax.experimental import pallas as pl
from jax.experimental.pallas import tpu as pltpu


def _prep_kernel(s_ref, n_ref, minv_ref):
    S = s_ref[...]
    n = n_ref[...]
    d = S.shape[0]
    ii = jax.lax.broadcasted_iota(jnp.int32, S.shape, 0)
    kk = jax.lax.broadcasted_iota(jnp.int32, S.shape, 1)
    eye3 = (ii == kk).astype(S.dtype)
    A = S / n[None, :, :] + 1e-6 * eye3
    Inv = eye3
    for j in range(d):
        rowA = A[j]
        inv_piv = 1.0 / rowA[j]
        rowA = rowA * inv_piv[None, :]
        rowI = Inv[j] * inv_piv[None, :]
        colA = A[:, j, :]
        is_row_j = ii == j
        A = jnp.where(is_row_j, rowA[None, :, :],
                      A - colA[:, None, :] * rowA[None, :, :])
        Inv = jnp.where(is_row_j, rowI[None, :, :],
                        Inv - colA[:, None, :] * rowI[None, :, :])
    minv_ref[...] = Inv.astype(jnp.bfloat16)


def _fused_kernel(xt_ref, mut_ref, cmp_ref, labt_ref, lst_ref, preds_ref,
                  clus_ref, g_scr):
    g = 4
    xT = xt_ref[...]
    bb = xT.shape[1]
    n_chunks = mut_ref.shape[0]
    n_groups = mut_ref.shape[2]
    gd = mut_ref.shape[1]
    CB = n_groups * g
    xt4 = jnp.concatenate([xT] * g, axis=0)
    rr = jax.lax.broadcasted_iota(jnp.int32, (gd, gd), 0) // (gd // g)
    cc = jax.lax.broadcasted_iota(jnp.int32, (gd, gd), 1) // (gd // g)
    bd_mask = rr == cc
    big = jnp.int32(2 ** 30)
    run_max = jnp.full((1, bb), -jnp.inf, dtype=jnp.float32)
    run_arg = jnp.zeros((1, bb), dtype=jnp.int32)
    ssum = jnp.zeros((1, bb), dtype=jnp.float32)
    idx_c = jax.lax.broadcasted_iota(jnp.int32, (CB, bb), 0)
    for jc in range(n_chunks):
        mu_blk = mut_ref[jc][...]
        cols = []
        for q in range(n_groups):
            diffT = xt4 - mu_blk[:, q][:, None]
            diffb = diffT.astype(jnp.bfloat16)
            cq = cmp_ref[jc * n_groups + q][...]
            bdq = jnp.where(bd_mask, jnp.concatenate([cq] * g, axis=1),
                            jnp.bfloat16(0))
            tT = jnp.dot(bdq, diffb, preferred_element_type=jnp.float32)
            prodT = tT * diffT
            cols.append(jnp.sum(prodT.reshape(g, gd // g, bb), axis=1))
        d2c = jnp.concatenate(cols, axis=0)
        Gc = jnp.exp(-0.5 * d2c)
        g_scr[jc * CB:(jc + 1) * CB, :] = Gc
        ssum = ssum + jnp.sum(Gc, axis=0, keepdims=True)
        cmax = jnp.max(Gc, axis=0, keepdims=True)
        carg = jnp.min(jnp.where(Gc == cmax, idx_c + jc * CB, big),
                       axis=0, keepdims=True)
        upd = cmax > run_max
        run_arg = jnp.where(upd, carg, run_arg)
        run_max = jnp.maximum(run_max, cmax)
    s = ssum + 1e-12
    labt = labt_ref[...]
    nc = labt.shape[0]
    gnb = (g_scr[...] / s).astype(jnp.bfloat16)
    lsT = jnp.dot(labt, gnb, preferred_element_type=jnp.float32)
    lsT = lsT / (jnp.sum(lsT, axis=0, keepdims=True) + 1e-12)
    lst_ref[...] = lsT
    pmax = jnp.max(lsT, axis=0, keepdims=True)
    idx_p = jax.lax.broadcasted_iota(jnp.int32, (nc, bb), 0)
    preds_ref[...] = jnp.min(jnp.where(lsT == pmax, idx_p, big),
                             axis=0, keepdims=True)
    clus_ref[...] = run_arg


@functools.partial(jax.jit, static_argnames=())
def kernel(data, mu, S, n, cluster_labels):
    B, D = data.shape
    C = mu.shape[0]
    NC = cluster_labels.shape[1]
    g = 4
    gd = g * D

    cb = 128
    minv_t = pl.pallas_call(
        _prep_kernel,
        grid=(C // cb,),
        in_specs=[
            pl.BlockSpec((D, D, cb), lambda i: (0, 0, i)),
            pl.BlockSpec((1, cb), lambda i: (0, i)),
        ],
        out_specs=pl.BlockSpec((D, D, cb), lambda i: (0, 0, i)),
        out_shape=jax.ShapeDtypeStruct((D, D, C), jnp.bfloat16),
    )(S.transpose(1, 2, 0), n.reshape(1, C))

    cmp = minv_t.transpose(2, 0, 1).reshape(C // g, gd, D)
    CB = 128
    mut = mu.reshape(C // g, gd).T.reshape(gd, C // CB, CB // g).transpose(1, 0, 2)
    dataT = data.T
    labt_bf = cluster_labels.T.astype(jnp.bfloat16)

    bb = 1024
    lsT, preds, clus = pl.pallas_call(
        _fused_kernel,
        grid=(B // bb,),
        in_specs=[
            pl.BlockSpec((D, bb), lambda i: (0, i)),
            pl.BlockSpec((C // CB, gd, CB // g), lambda i: (0, 0, 0)),
            pl.BlockSpec((C // g, gd, D), lambda i: (0, 0, 0)),
            pl.BlockSpec((NC, C), lambda i: (0, 0)),
        ],
        out_specs=[
            pl.BlockSpec((NC, bb), lambda i: (0, i)),
            pl.BlockSpec((1, bb), lambda i: (0, i)),
            pl.BlockSpec((1, bb), lambda i: (0, i)),
        ],
        out_shape=[
            jax.ShapeDtypeStruct((NC, B), jnp.float32),
            jax.ShapeDtypeStruct((1, B), jnp.int32),
            jax.ShapeDtypeStruct((1, B), jnp.int32),
        ],
        scratch_shapes=[pltpu.VMEM((C, bb), jnp.float32)],
    )(dataT, mut, cmp, labt_bf)

    return lsT.T, preds.reshape(B), clus.reshape(B)


# speedup vs baseline: 1.6534x; 1.0139x over previous
import functools

import jax
import jax.numpy as jnp
from jax.experimental import pallas as pl
from jax.experimental.pallas import tpu as pltpu


def _prep_kernel(s_ref, n_ref, minv_ref):
    S = s_ref[...]
    n = n_ref[...]
    d = S.shape[0]
    ii = jax.lax.broadcasted_iota(jnp.int32, S.shape, 0)
    kk = jax.lax.broadcasted_iota(jnp.int32, S.shape, 1)
    eye3 = (ii == kk).astype(S.dtype)
    A = S / n[None, :, :] + 1e-6 * eye3
    Inv = eye3
    for j in range(d):
        rowA = A[j]
        inv_piv = 1.0 / rowA[j]
        rowA = rowA * inv_piv[None, :]
        rowI = Inv[j] * inv_piv[None, :]
        colA = A[:, j, :]
        is_row_j = ii == j
        A = jnp.where(is_row_j, rowA[None, :, :],
                      A - colA[:, None, :] * rowA[None, :, :])
        Inv = jnp.where(is_row_j, rowI[None, :, :],
                        Inv - colA[:, None, :] * rowI[None, :, :])
    minv_ref[...] = Inv.astype(jnp.bfloat16)


def _fused_kernel(xt_ref, mut_ref, cmp_ref, labt_ref, lst_ref, preds_ref,
                  clus_ref, g_scr):
    g = 4
    xT = xt_ref[...]
    bb = xT.shape[1]
    n_chunks = mut_ref.shape[0]
    n_groups = mut_ref.shape[2]
    gd = mut_ref.shape[1]
    CB = n_groups * g
    xt4 = jnp.concatenate([xT] * g, axis=0)
    rr = jax.lax.broadcasted_iota(jnp.int32, (gd, gd), 0) // (gd // g)
    cc = jax.lax.broadcasted_iota(jnp.int32, (gd, gd), 1) // (gd // g)
    bd_mask = rr == cc
    big = jnp.int32(2 ** 30)
    run_max = jnp.full((1, bb), -jnp.inf, dtype=jnp.float32)
    run_arg = jnp.zeros((1, bb), dtype=jnp.int32)
    ssum = jnp.zeros((1, bb), dtype=jnp.float32)
    idx_c = jax.lax.broadcasted_iota(jnp.int32, (CB, bb), 0)
    for jc in range(n_chunks):
        mu_blk = mut_ref[jc][...]
        cols = []
        for q in range(n_groups):
            diffT = xt4 - mu_blk[:, q][:, None]
            diffb = diffT.astype(jnp.bfloat16)
            cq = cmp_ref[jc * n_groups + q][...]
            bdq = jnp.where(bd_mask, jnp.concatenate([cq] * g, axis=1),
                            jnp.bfloat16(0))
            tT = jnp.dot(bdq, diffb, preferred_element_type=jnp.float32)
            prodT = tT * diffT
            cols.append(jnp.sum(prodT.reshape(g, gd // g, bb), axis=1))
        d2c = jnp.concatenate(cols, axis=0)
        Gc = jnp.exp(-0.5 * d2c)
        g_scr[jc * CB:(jc + 1) * CB, :] = Gc
        ssum = ssum + jnp.sum(Gc, axis=0, keepdims=True)
        cmax = jnp.max(Gc, axis=0, keepdims=True)
        carg = jnp.min(jnp.where(Gc == cmax, idx_c + jc * CB, big),
                       axis=0, keepdims=True)
        upd = cmax > run_max
        run_arg = jnp.where(upd, carg, run_arg)
        run_max = jnp.maximum(run_max, cmax)
    s = ssum + 1e-12
    labt = labt_ref[...]
    nc = labt.shape[0]
    gnb = (g_scr[...] / s).astype(jnp.bfloat16)
    lsT = jnp.dot(labt, gnb, preferred_element_type=jnp.float32)
    lsT = lsT / (jnp.sum(lsT, axis=0, keepdims=True) + 1e-12)
    lst_ref[...] = lsT
    pmax = jnp.max(lsT, axis=0, keepdims=True)
    idx_p = jax.lax.broadcasted_iota(jnp.int32, (nc, bb), 0)
    preds_ref[...] = jnp.min(jnp.where(lsT == pmax, idx_p, big),
                             axis=0, keepdims=True)
    clus_ref[...] = run_arg


@functools.partial(jax.jit, static_argnames=())
def kernel(data, mu, S, n, cluster_labels):
    B, D = data.shape
    C = mu.shape[0]
    NC = cluster_labels.shape[1]
    g = 4
    gd = g * D

    cb = 128
    minv_t = pl.pallas_call(
        _prep_kernel,
        grid=(C // cb,),
        in_specs=[
            pl.BlockSpec((D, D, cb), lambda i: (0, 0, i)),
            pl.BlockSpec((1, cb), lambda i: (0, i)),
        ],
        out_specs=pl.BlockSpec((D, D, cb), lambda i: (0, 0, i)),
        out_shape=jax.ShapeDtypeStruct((D, D, C), jnp.bfloat16),
    )(S.transpose(1, 2, 0), n.reshape(1, C))

    cmp = minv_t.transpose(2, 0, 1).reshape(C // g, gd, D)
    CB = 128
    mut = mu.reshape(C // g, gd).T.reshape(gd, C // CB, CB // g).transpose(1, 0, 2)
    dataT = data.T
    labt_bf = cluster_labels.T.astype(jnp.bfloat16)

    bb = 2048
    lsT, preds, clus = pl.pallas_call(
        _fused_kernel,
        grid=(B // bb,),
        in_specs=[
            pl.BlockSpec((D, bb), lambda i: (0, i)),
            pl.BlockSpec((C // CB, gd, CB // g), lambda i: (0, 0, 0)),
            pl.BlockSpec((C // g, gd, D), lambda i: (0, 0, 0)),
            pl.BlockSpec((NC, C), lambda i: (0, 0)),
        ],
        out_specs=[
            pl.BlockSpec((NC, bb), lambda i: (0, i)),
            pl.BlockSpec((1, bb), lambda i: (0, i)),
            pl.BlockSpec((1, bb), lambda i: (0, i)),
        ],
        out_shape=[
            jax.ShapeDtypeStruct((NC, B), jnp.float32),
            jax.ShapeDtypeStruct((1, B), jnp.int32),
            jax.ShapeDtypeStruct((1, B), jnp.int32),
        ],
        scratch_shapes=[pltpu.VMEM((C, bb), jnp.float32)],
    )(dataT, mut, cmp, labt_bf)

    return lsT.T, preds.reshape(B), clus.reshape(B)
